# baseline (speedup 1.0000x reference)
import jax
import jax.numpy as jnp
from jax.experimental import pallas as pl
from jax.experimental.pallas import tpu as pltpu


def _body(mem_ref, h_ref, r1_ref, r2_ref, fil_ref, wg_ref, bg_ref, out_ref):
    m = mem_ref[0]
    r1 = r1_ref[0]
    r2 = r2_ref[0]
    fil = fil_ref[0]

    M = m.shape[0]

    h = h_ref[0]
    wg = wg_ref[...]
    dot = jnp.sum(h * wg, axis=1, keepdims=True)
    gate = jax.nn.sigmoid(dot + bg_ref[0, 0] + 1.0)

    tmp = jnp.sum(m * r1, axis=0)
    prev = jnp.sum(tmp * r2, axis=0, keepdims=True)

    u = gate * (fil - prev) * (1.0 / M)

    a2 = r2 * u
    new = m + r1 * a2[None, :, :]

    n2 = jnp.sum(new * new, keepdims=True)
    inv = jnp.minimum(jax.lax.rsqrt(n2), 1.0)
    out_ref[0] = new * inv


def kernel(memory_state, hidden_state, role1, role2, filer, W_gate, b_gate):
    B, M = memory_state.shape[0], memory_state.shape[1]
    H = hidden_state.shape[1]

    h3 = hidden_state.reshape(B, 1, H)
    r1v = role1.reshape(B, M, 1, 1)
    r2v = role2.reshape(B, M, 1)
    fil3 = filer.reshape(B, 1, M)
    bg = b_gate.reshape(1, 1)

    out = pl.pallas_call(
        _body,
        grid=(B,),
        in_specs=[
            pl.BlockSpec((1, M, M, M), lambda i: (i, 0, 0, 0)),
            pl.BlockSpec((1, 1, H), lambda i: (i, 0, 0)),
            pl.BlockSpec((1, M, 1, 1), lambda i: (i, 0, 0, 0)),
            pl.BlockSpec((1, M, 1), lambda i: (i, 0, 0)),
            pl.BlockSpec((1, 1, M), lambda i: (i, 0, 0)),
            pl.BlockSpec((1, H), lambda i: (0, 0)),
            pl.BlockSpec((1, 1), lambda i: (0, 0)),
        ],
        out_specs=pl.BlockSpec((1, M, M, M), lambda i: (i, 0, 0, 0)),
        out_shape=jax.ShapeDtypeStruct((B, M, M, M), jnp.float32),
        compiler_params=pltpu.CompilerParams(
            dimension_semantics=("parallel",),
        ),
    )(memory_state, h3, r1v, r2v, fil3, W_gate, bg)
    return out


# speedup vs baseline: 1.6296x; 1.6296x over previous
import jax
import jax.numpy as jnp
from jax.experimental import pallas as pl
from jax.experimental.pallas import tpu as pltpu


def _body(mem_ref, aux_ref, r2e_ref, wp_ref, out_ref):
    m2 = mem_ref[0]
    aux = aux_ref[0]
    wp = wp_ref[0]
    r2e = r2e_ref[0]

    M = m2.shape[0]

    gd = jnp.sum(aux * wp, axis=(0, 1), keepdims=True)[:, :1]
    gate = jax.nn.sigmoid(gd + 1.0)

    r1row = aux[0:1, 0:M]
    fil = aux[2:3, 0:M]

    s = jnp.dot(r1row, m2, preferred_element_type=jnp.float32)
    z = s * r2e
    w = (M * M) // 2
    while w >= M:
        z = z[:, :w] + z[:, w:2 * w]
        w //= 2
    prev = z

    u = gate * (fil - prev) * (1.0 / M)

    u2 = jnp.concatenate([u, u], axis=1)
    ut = pltpu.repeat(u2, (M * M) // (2 * M), axis=1)
    v = r2e * ut

    r1col = jnp.transpose(r1row)
    new = m2 + r1col * v

    ssq = jnp.sum(new * new, axis=0, keepdims=True)
    n2 = jnp.sum(ssq, axis=1, keepdims=True)
    inv = jnp.minimum(jax.lax.rsqrt(n2), 1.0)
    out_ref[0] = new * inv


def kernel(memory_state, hidden_state, role1, role2, filer, W_gate, b_gate):
    B, M = memory_state.shape[0], memory_state.shape[1]
    H = hidden_state.shape[1]

    mem2 = memory_state.reshape(B, M, M * M)

    def row(x):
        return jnp.pad(x.reshape(B, 1, M), ((0, 0), (0, 0), (0, 128 - M)))

    aux = jnp.concatenate(
        [row(role1), row(role2), row(filer),
         hidden_state.reshape(B, 4, 128),
         jnp.broadcast_to(b_gate.reshape(1, 1, 1), (B, 1, 128))], axis=1)

    wp = jnp.concatenate(
        [jnp.zeros((1, 3, 128), jnp.float32),
         W_gate.reshape(1, 4, 128),
         jnp.zeros((1, 1, 128), jnp.float32).at[0, 0, 0].set(1.0)], axis=1)

    r2e = jnp.repeat(role2, M, axis=1).reshape(B, 1, M * M)

    out = pl.pallas_call(
        _body,
        grid=(B,),
        in_specs=[
            pl.BlockSpec((1, M, M * M), lambda i: (i, 0, 0)),
            pl.BlockSpec((1, 8, 128), lambda i: (i, 0, 0)),
            pl.BlockSpec((1, 1, M * M), lambda i: (i, 0, 0)),
            pl.BlockSpec((1, 8, 128), lambda i: (0, 0, 0)),
        ],
        out_specs=pl.BlockSpec((1, M, M * M), lambda i: (i, 0, 0)),
        out_shape=jax.ShapeDtypeStruct((B, M, M * M), jnp.float32),
        compiler_params=pltpu.CompilerParams(
            dimension_semantics=("parallel",),
        ),
    )(mem2, aux, r2e, wp)
    return out.reshape(B, M, M, M)


# speedup vs baseline: 4.0914x; 2.5106x over previous
import jax
import jax.numpy as jnp
from jax.experimental import pallas as pl
from jax.experimental.pallas import tpu as pltpu


def _reduce_body(mem_ref, r1_ref, r2_ref, prev_ref, sq_ref):
    r = pl.program_id(1)
    slab = mem_ref[0]
    r2b = r2_ref[...]
    q = jnp.sum(slab * r2b, axis=0)
    contrib = r1_ref[0] * q
    sqc = jnp.sum(slab * slab, axis=(0, 1), keepdims=True)[0]

    @pl.when(r == 0)
    def _():
        prev_ref[...] = contrib
        sq_ref[...] = sqc[None]

    @pl.when(r != 0)
    def _():
        prev_ref[...] += contrib
        sq_ref[...] += sqc[None]


def _finalize_body(prev_ref, sq_ref, r1_ref, r2_ref, fil_ref, h_ref, w_ref,
                   bg_ref, u_ref, inv_ref):
    prev = prev_ref[...]
    M = prev.shape[0]
    gd = jnp.sum(h_ref[...] * w_ref[...], axis=0, keepdims=True)
    gate = jax.nn.sigmoid(gd + bg_ref[0, 0] + 1.0)
    u = gate * (fil_ref[...] - prev) * (1.0 / M)
    pu = jnp.sum(prev * u, axis=0, keepdims=True)
    s1 = jnp.sum(r1_ref[...] * r1_ref[...], axis=0, keepdims=True)
    s2 = jnp.sum(r2_ref[...] * r2_ref[...], axis=0, keepdims=True)
    su = jnp.sum(u * u, axis=0, keepdims=True)
    sq = sq_ref[0] + sq_ref[1]
    n2 = sq + 2.0 * pu + s1 * s2 * su
    inv = jnp.minimum(jax.lax.rsqrt(n2), 1.0)
    u_ref[...] = u
    inv_ref[...] = inv


def _update_body(mem_ref, r1_ref, r2_ref, u_ref, inv_ref, out_ref):
    slab = mem_ref[0]
    a = r2_ref[...] * u_ref[...][None, :, :]
    out_ref[0] = (slab + r1_ref[0] * a) * inv_ref[...]


def kernel(memory_state, hidden_state, role1, role2, filer, W_gate, b_gate):
    B, M = memory_state.shape[0], memory_state.shape[1]
    H = hidden_state.shape[1]
    f32 = jnp.float32

    mem_t = jnp.transpose(memory_state, (1, 2, 3, 0))
    r1t = jnp.transpose(role1)
    r2t = jnp.transpose(role2)
    fil_t = jnp.transpose(filer)
    r1t3 = r1t.reshape(M, 1, B)
    r2t3 = r2t.reshape(M, 1, B)
    h_t = jnp.transpose(hidden_state)
    wcol = W_gate.reshape(H, 1)
    bg = b_gate.reshape(1, 1)

    Mh = M // 2
    prev, sqp = pl.pallas_call(
        _reduce_body,
        grid=(2, M),
        in_specs=[
            pl.BlockSpec((1, M, Mh, B), lambda fh, r: (r, 0, fh, 0)),
            pl.BlockSpec((1, 1, B), lambda fh, r: (r, 0, 0)),
            pl.BlockSpec((M, 1, B), lambda fh, r: (0, 0, 0)),
        ],
        out_specs=[
            pl.BlockSpec((Mh, B), lambda fh, r: (fh, 0)),
            pl.BlockSpec((1, 1, B), lambda fh, r: (fh, 0, 0)),
        ],
        out_shape=[
            jax.ShapeDtypeStruct((M, B), f32),
            jax.ShapeDtypeStruct((2, 1, B), f32),
        ],
        compiler_params=pltpu.CompilerParams(
            dimension_semantics=("parallel", "arbitrary"),
        ),
    )(mem_t, r1t3, r2t3)

    u, inv = pl.pallas_call(
        _finalize_body,
        grid=(1,),
        in_specs=[
            pl.BlockSpec((M, B), lambda i: (0, 0)),
            pl.BlockSpec((2, 1, B), lambda i: (0, 0, 0)),
            pl.BlockSpec((M, B), lambda i: (0, 0)),
            pl.BlockSpec((M, B), lambda i: (0, 0)),
            pl.BlockSpec((M, B), lambda i: (0, 0)),
            pl.BlockSpec((H, B), lambda i: (0, 0)),
            pl.BlockSpec((H, 1), lambda i: (0, 0)),
            pl.BlockSpec((1, 1), lambda i: (0, 0)),
        ],
        out_specs=[
            pl.BlockSpec((M, B), lambda i: (0, 0)),
            pl.BlockSpec((1, B), lambda i: (0, 0)),
        ],
        out_shape=[
            jax.ShapeDtypeStruct((M, B), f32),
            jax.ShapeDtypeStruct((1, B), f32),
        ],
        compiler_params=pltpu.CompilerParams(
            dimension_semantics=("arbitrary",),
        ),
    )(prev, sqp, r1t, r2t, fil_t, h_t, wcol, bg)

    out_t = pl.pallas_call(
        _update_body,
        grid=(M,),
        in_specs=[
            pl.BlockSpec((1, M, M, B), lambda r: (r, 0, 0, 0)),
            pl.BlockSpec((1, 1, B), lambda r: (r, 0, 0)),
            pl.BlockSpec((M, 1, B), lambda r: (0, 0, 0)),
            pl.BlockSpec((M, B), lambda r: (0, 0)),
            pl.BlockSpec((1, B), lambda r: (0, 0)),
        ],
        out_specs=pl.BlockSpec((1, M, M, B), lambda r: (r, 0, 0, 0)),
        out_shape=jax.ShapeDtypeStruct((M, M, M, B), f32),
        compiler_params=pltpu.CompilerParams(
            dimension_semantics=("parallel",),
        ),
    )(mem_t, r1t3, r2t3, u, inv)

    return jnp.transpose(out_t, (3, 0, 1, 2))


# speedup vs baseline: 4.6497x; 1.1365x over previous
import jax
import jax.numpy as jnp
from jax.experimental import pallas as pl
from jax.experimental.pallas import tpu as pltpu


def _reduce_body(mem_ref, r1_ref, r2_ref, prev_ref, sq_ref):
    rr = pl.program_id(1)
    slab = mem_ref[0]
    r2b = r2_ref[...]
    q = jnp.sum(slab * r2b, axis=0)
    contrib = r1_ref[0] * q
    sqc = jnp.sum(slab * slab, axis=(0, 1), keepdims=True)[0]

    @pl.when(rr == 0)
    def _():
        prev_ref[0] = contrib
        sq_ref[...] = sqc[None]

    @pl.when(rr != 0)
    def _():
        prev_ref[0] += contrib
        sq_ref[...] += sqc[None]


def _finalize_body(prev_ref, sq_ref, r1_ref, r2_ref, fil_ref, h_ref, w_ref,
                   bg_ref, u_ref, inv_ref):
    prev = prev_ref[0] + prev_ref[1]
    M = prev.shape[0]
    gd = jnp.sum(h_ref[...] * w_ref[...], axis=0, keepdims=True)
    gate = jax.nn.sigmoid(gd + bg_ref[0, 0] + 1.0)
    u = gate * (fil_ref[...] - prev) * (1.0 / M)
    pu = jnp.sum(prev * u, axis=0, keepdims=True)
    s1 = jnp.sum(r1_ref[...] * r1_ref[...], axis=0, keepdims=True)
    s2 = jnp.sum(r2_ref[...] * r2_ref[...], axis=0, keepdims=True)
    su = jnp.sum(u * u, axis=0, keepdims=True)
    sq = sq_ref[0] + sq_ref[1]
    n2 = sq + 2.0 * pu + s1 * s2 * su
    inv = jnp.minimum(jax.lax.rsqrt(n2), 1.0)
    u_ref[...] = u
    inv_ref[...] = inv


def _update_body(mem_ref, r1_ref, r2_ref, u_ref, inv_ref, out_ref):
    slab = mem_ref[0]
    a = r2_ref[...] * u_ref[...][None, :, :]
    out_ref[0] = (slab + r1_ref[0] * a) * inv_ref[...]


def kernel(memory_state, hidden_state, role1, role2, filer, W_gate, b_gate):
    B, M = memory_state.shape[0], memory_state.shape[1]
    H = hidden_state.shape[1]
    f32 = jnp.float32

    mem_t = jnp.transpose(memory_state, (1, 2, 3, 0))
    r1t = jnp.transpose(role1)
    r2t = jnp.transpose(role2)
    fil_t = jnp.transpose(filer)
    r1t3 = r1t.reshape(M, 1, B)
    r2t3 = r2t.reshape(M, 1, B)
    h_t = jnp.transpose(hidden_state)
    wcol = W_gate.reshape(H, 1)
    bg = b_gate.reshape(1, 1)

    Mr = M // 2
    prev2, sqp = pl.pallas_call(
        _reduce_body,
        grid=(2, Mr),
        in_specs=[
            pl.BlockSpec((1, M, M, B), lambda rh, rr: (rh * Mr + rr, 0, 0, 0)),
            pl.BlockSpec((1, 1, B), lambda rh, rr: (rh * Mr + rr, 0, 0)),
            pl.BlockSpec((M, 1, B), lambda rh, rr: (0, 0, 0)),
        ],
        out_specs=[
            pl.BlockSpec((1, M, B), lambda rh, rr: (rh, 0, 0)),
            pl.BlockSpec((1, 1, B), lambda rh, rr: (rh, 0, 0)),
        ],
        out_shape=[
            jax.ShapeDtypeStruct((2, M, B), f32),
            jax.ShapeDtypeStruct((2, 1, B), f32),
        ],
        compiler_params=pltpu.CompilerParams(
            dimension_semantics=("parallel", "arbitrary"),
        ),
    )(mem_t, r1t3, r2t3)

    u, inv = pl.pallas_call(
        _finalize_body,
        grid=(1,),
        in_specs=[
            pl.BlockSpec((2, M, B), lambda i: (0, 0, 0)),
            pl.BlockSpec((2, 1, B), lambda i: (0, 0, 0)),
            pl.BlockSpec((M, B), lambda i: (0, 0)),
            pl.BlockSpec((M, B), lambda i: (0, 0)),
            pl.BlockSpec((M, B), lambda i: (0, 0)),
            pl.BlockSpec((H, B), lambda i: (0, 0)),
            pl.BlockSpec((H, 1), lambda i: (0, 0)),
            pl.BlockSpec((1, 1), lambda i: (0, 0)),
        ],
        out_specs=[
            pl.BlockSpec((M, B), lambda i: (0, 0)),
            pl.BlockSpec((1, B), lambda i: (0, 0)),
        ],
        out_shape=[
            jax.ShapeDtypeStruct((M, B), f32),
            jax.ShapeDtypeStruct((1, B), f32),
        ],
        compiler_params=pltpu.CompilerParams(
            dimension_semantics=("arbitrary",),
        ),
    )(prev2, sqp, r1t, r2t, fil_t, h_t, wcol, bg)

    out_t = pl.pallas_call(
        _update_body,
        grid=(M,),
        in_specs=[
            pl.BlockSpec((1, M, M, B), lambda r: (r, 0, 0, 0)),
            pl.BlockSpec((1, 1, B), lambda r: (r, 0, 0)),
            pl.BlockSpec((M, 1, B), lambda r: (0, 0, 0)),
            pl.BlockSpec((M, B), lambda r: (0, 0)),
            pl.BlockSpec((1, B), lambda r: (0, 0)),
        ],
        out_specs=pl.BlockSpec((1, M, M, B), lambda r: (r, 0, 0, 0)),
        out_shape=jax.ShapeDtypeStruct((M, M, M, B), f32),
        compiler_params=pltpu.CompilerParams(
            dimension_semantics=("parallel",),
        ),
    )(mem_t, r1t3, r2t3, u, inv)

    return jnp.transpose(out_t, (3, 0, 1, 2))


# speedup vs baseline: 4.9997x; 1.0753x over previous
import jax
import jax.numpy as jnp
from jax.experimental import pallas as pl
from jax.experimental.pallas import tpu as pltpu


def _reduce_body(mem_ref, r1_ref, r2_ref, prev_ref, sq_ref):
    rr = pl.program_id(1)
    slab = mem_ref[...]
    r1b = r1_ref[...]
    r2b = r2_ref[...]
    q = jnp.sum(slab * r2b[None], axis=1)
    contrib = jnp.sum(q * r1b, axis=0)
    sqc = jnp.sum(slab * slab, axis=(0, 1, 2), keepdims=True)[0, 0]

    @pl.when(rr == 0)
    def _():
        prev_ref[0] = contrib
        sq_ref[...] = sqc[None]

    @pl.when(rr != 0)
    def _():
        prev_ref[0] += contrib
        sq_ref[...] += sqc[None]


def _finalize_body(prev_ref, sq_ref, r1_ref, r2_ref, fil_ref, h_ref, w_ref,
                   bg_ref, u_ref, inv_ref):
    prev = prev_ref[0] + prev_ref[1]
    M = prev.shape[0]
    gd = jnp.sum(h_ref[...] * w_ref[...], axis=0, keepdims=True)
    gate = jax.nn.sigmoid(gd + bg_ref[0, 0] + 1.0)
    u = gate * (fil_ref[...] - prev) * (1.0 / M)
    pu = jnp.sum(prev * u, axis=0, keepdims=True)
    s1 = jnp.sum(r1_ref[...] * r1_ref[...], axis=0, keepdims=True)
    s2 = jnp.sum(r2_ref[...] * r2_ref[...], axis=0, keepdims=True)
    su = jnp.sum(u * u, axis=0, keepdims=True)
    sq = sq_ref[0] + sq_ref[1]
    n2 = sq + 2.0 * pu + s1 * s2 * su
    inv = jnp.minimum(jax.lax.rsqrt(n2), 1.0)
    u_ref[...] = u
    inv_ref[...] = inv


def _update_body(mem_ref, r1_ref, r2_ref, u_ref, inv_ref, out_ref):
    slab = mem_ref[...]
    a = r2_ref[...] * u_ref[...][None, :, :]
    r1b = r1_ref[...][:, None]
    out_ref[...] = (slab + r1b * a[None]) * inv_ref[...]


def kernel(memory_state, hidden_state, role1, role2, filer, W_gate, b_gate):
    B, M = memory_state.shape[0], memory_state.shape[1]
    H = hidden_state.shape[1]
    f32 = jnp.float32

    mem_t = jnp.transpose(memory_state, (1, 2, 3, 0))
    r1t = jnp.transpose(role1)
    r2t = jnp.transpose(role2)
    fil_t = jnp.transpose(filer)
    r1t3 = r1t.reshape(M, 1, B)
    r2t3 = r2t.reshape(M, 1, B)
    h_t = jnp.transpose(hidden_state)
    wcol = W_gate.reshape(H, 1)
    bg = b_gate.reshape(1, 1)

    R = 2
    Mr = M // (2 * R)
    prev2, sqp = pl.pallas_call(
        _reduce_body,
        grid=(2, Mr),
        in_specs=[
            pl.BlockSpec((R, M, M, B), lambda rh, rr: (rh * Mr + rr, 0, 0, 0)),
            pl.BlockSpec((R, 1, B), lambda rh, rr: (rh * Mr + rr, 0, 0)),
            pl.BlockSpec((M, 1, B), lambda rh, rr: (0, 0, 0)),
        ],
        out_specs=[
            pl.BlockSpec((1, M, B), lambda rh, rr: (rh, 0, 0)),
            pl.BlockSpec((1, 1, B), lambda rh, rr: (rh, 0, 0)),
        ],
        out_shape=[
            jax.ShapeDtypeStruct((2, M, B), f32),
            jax.ShapeDtypeStruct((2, 1, B), f32),
        ],
        compiler_params=pltpu.CompilerParams(
            dimension_semantics=("parallel", "arbitrary"),
        ),
    )(mem_t, r1t3, r2t3)

    u, inv = pl.pallas_call(
        _finalize_body,
        grid=(1,),
        in_specs=[
            pl.BlockSpec((2, M, B), lambda i: (0, 0, 0)),
            pl.BlockSpec((2, 1, B), lambda i: (0, 0, 0)),
            pl.BlockSpec((M, B), lambda i: (0, 0)),
            pl.BlockSpec((M, B), lambda i: (0, 0)),
            pl.BlockSpec((M, B), lambda i: (0, 0)),
            pl.BlockSpec((H, B), lambda i: (0, 0)),
            pl.BlockSpec((H, 1), lambda i: (0, 0)),
            pl.BlockSpec((1, 1), lambda i: (0, 0)),
        ],
        out_specs=[
            pl.BlockSpec((M, B), lambda i: (0, 0)),
            pl.BlockSpec((1, B), lambda i: (0, 0)),
        ],
        out_shape=[
            jax.ShapeDtypeStruct((M, B), f32),
            jax.ShapeDtypeStruct((1, B), f32),
        ],
        compiler_params=pltpu.CompilerParams(
            dimension_semantics=("arbitrary",),
        ),
    )(prev2, sqp, r1t, r2t, fil_t, h_t, wcol, bg)

    R2 = 2
    out_t = pl.pallas_call(
        _update_body,
        grid=(M // R2,),
        in_specs=[
            pl.BlockSpec((R2, M, M, B), lambda r: (r, 0, 0, 0)),
            pl.BlockSpec((R2, 1, B), lambda r: (r, 0, 0)),
            pl.BlockSpec((M, 1, B), lambda r: (0, 0, 0)),
            pl.BlockSpec((M, B), lambda r: (0, 0)),
            pl.BlockSpec((1, B), lambda r: (0, 0)),
        ],
        out_specs=pl.BlockSpec((R2, M, M, B), lambda r: (r, 0, 0, 0)),
        out_shape=jax.ShapeDtypeStruct((M, M, M, B), f32),
        compiler_params=pltpu.CompilerParams(
            dimension_semantics=("parallel",),
            vmem_limit_bytes=56 * 1024 * 1024,
        ),
    )(mem_t, r1t3, r2t3, u, inv)

    return jnp.transpose(out_t, (3, 0, 1, 2))


# speedup vs baseline: 5.1366x; 1.0274x over previous
import jax
import jax.numpy as jnp
from jax.experimental import pallas as pl
from jax.experimental.pallas import tpu as pltpu


def _reduce_body(mem_ref, r1_ref, r2_ref, prev_ref, sq_ref):
    rr = pl.program_id(1)
    slab = mem_ref[...]
    r1b = r1_ref[...]
    r2b = r2_ref[...]
    q = jnp.sum(slab * r2b[None], axis=1)
    contrib = jnp.sum(q * r1b, axis=0)
    sqc = jnp.sum(slab * slab, axis=(0, 1, 2), keepdims=True)[0, 0]

    @pl.when(rr == 0)
    def _():
        prev_ref[0] = contrib
        sq_ref[...] = sqc[None]

    @pl.when(rr != 0)
    def _():
        prev_ref[0] += contrib
        sq_ref[...] += sqc[None]


def _finalize_body(prev_ref, sq_ref, r1_ref, r2_ref, fil_ref, h_ref, w_ref,
                   bg_ref, u_ref, inv_ref):
    prev = prev_ref[0] + prev_ref[1]
    M = prev.shape[0]
    gd = jnp.sum(h_ref[...] * w_ref[...], axis=0, keepdims=True)
    gate = jax.nn.sigmoid(gd + bg_ref[0, 0] + 1.0)
    u = gate * (fil_ref[...] - prev) * (1.0 / M)
    pu = jnp.sum(prev * u, axis=0, keepdims=True)
    s1 = jnp.sum(r1_ref[...] * r1_ref[...], axis=0, keepdims=True)
    s2 = jnp.sum(r2_ref[...] * r2_ref[...], axis=0, keepdims=True)
    su = jnp.sum(u * u, axis=0, keepdims=True)
    sq = sq_ref[0] + sq_ref[1]
    n2 = sq + 2.0 * pu + s1 * s2 * su
    inv = jnp.minimum(jax.lax.rsqrt(n2), 1.0)
    u_ref[...] = u
    inv_ref[...] = inv


def _update_body(mem_ref, r1_ref, r2_ref, u_ref, inv_ref, out_ref):
    slab = mem_ref[...]
    a = r2_ref[...] * u_ref[...][None, :, :]
    r1b = r1_ref[...][:, None]
    out_ref[...] = (slab + r1b * a[None]) * inv_ref[...]


def kernel(memory_state, hidden_state, role1, role2, filer, W_gate, b_gate):
    B, M = memory_state.shape[0], memory_state.shape[1]
    H = hidden_state.shape[1]
    f32 = jnp.float32

    mem_t = jnp.transpose(memory_state, (1, 2, 3, 0))
    r1t = jnp.transpose(role1)
    r2t = jnp.transpose(role2)
    fil_t = jnp.transpose(filer)
    r1t3 = r1t.reshape(M, 1, B)
    r2t3 = r2t.reshape(M, 1, B)
    h_t = jnp.transpose(hidden_state)
    wcol = W_gate.reshape(H, 1)
    bg = b_gate.reshape(1, 1)

    R = 4
    Mr = M // (2 * R)
    prev2, sqp = pl.pallas_call(
        _reduce_body,
        grid=(2, Mr),
        in_specs=[
            pl.BlockSpec((R, M, M, B), lambda rh, rr: (rh * Mr + rr, 0, 0, 0)),
            pl.BlockSpec((R, 1, B), lambda rh, rr: (rh * Mr + rr, 0, 0)),
            pl.BlockSpec((M, 1, B), lambda rh, rr: (0, 0, 0)),
        ],
        out_specs=[
            pl.BlockSpec((1, M, B), lambda rh, rr: (rh, 0, 0)),
            pl.BlockSpec((1, 1, B), lambda rh, rr: (rh, 0, 0)),
        ],
        out_shape=[
            jax.ShapeDtypeStruct((2, M, B), f32),
            jax.ShapeDtypeStruct((2, 1, B), f32),
        ],
        compiler_params=pltpu.CompilerParams(
            dimension_semantics=("parallel", "arbitrary"),
        ),
    )(mem_t, r1t3, r2t3)

    u, inv = pl.pallas_call(
        _finalize_body,
        grid=(1,),
        in_specs=[
            pl.BlockSpec((2, M, B), lambda i: (0, 0, 0)),
            pl.BlockSpec((2, 1, B), lambda i: (0, 0, 0)),
            pl.BlockSpec((M, B), lambda i: (0, 0)),
            pl.BlockSpec((M, B), lambda i: (0, 0)),
            pl.BlockSpec((M, B), lambda i: (0, 0)),
            pl.BlockSpec((H, B), lambda i: (0, 0)),
            pl.BlockSpec((H, 1), lambda i: (0, 0)),
            pl.BlockSpec((1, 1), lambda i: (0, 0)),
        ],
        out_specs=[
            pl.BlockSpec((M, B), lambda i: (0, 0)),
            pl.BlockSpec((1, B), lambda i: (0, 0)),
        ],
        out_shape=[
            jax.ShapeDtypeStruct((M, B), f32),
            jax.ShapeDtypeStruct((1, B), f32),
        ],
        compiler_params=pltpu.CompilerParams(
            dimension_semantics=("arbitrary",),
        ),
    )(prev2, sqp, r1t, r2t, fil_t, h_t, wcol, bg)

    R2 = 2
    out_t = pl.pallas_call(
        _update_body,
        grid=(M // R2,),
        in_specs=[
            pl.BlockSpec((R2, M, M, B), lambda r: (r, 0, 0, 0)),
            pl.BlockSpec((R2, 1, B), lambda r: (r, 0, 0)),
            pl.BlockSpec((M, 1, B), lambda r: (0, 0, 0)),
            pl.BlockSpec((M, B), lambda r: (0, 0)),
            pl.BlockSpec((1, B), lambda r: (0, 0)),
        ],
        out_specs=pl.BlockSpec((R2, M, M, B), lambda r: (r, 0, 0, 0)),
        out_shape=jax.ShapeDtypeStruct((M, M, M, B), f32),
        compiler_params=pltpu.CompilerParams(
            dimension_semantics=("parallel",),
            vmem_limit_bytes=56 * 1024 * 1024,
        ),
    )(mem_t, r1t3, r2t3, u, inv)

    return jnp.transpose(out_t, (3, 0, 1, 2))


# speedup vs baseline: 5.1539x; 1.0034x over previous
import jax
import jax.numpy as jnp
from jax.experimental import pallas as pl
from jax.experimental.pallas import tpu as pltpu


def _reduce_body(mem_ref, r1_ref, r2_ref, prev_ref, sq_ref):
    rr = pl.program_id(0)
    slab = mem_ref[...]
    r1b = r1_ref[...]
    r2b = r2_ref[...][:, None, :]
    q = jnp.sum(slab * r2b[None], axis=1)
    contrib = jnp.sum(q * r1b, axis=0)
    sqc = jnp.sum(slab * slab, axis=(0, 1, 2), keepdims=True)[0, 0]

    @pl.when(rr == 0)
    def _():
        prev_ref[...] = contrib
        sq_ref[...] = sqc
    @pl.when(rr != 0)
    def _():
        prev_ref[...] += contrib
        sq_ref[...] += sqc


def _finalize_body(prev_ref, sq_ref, r1_ref, r2_ref, fil_ref, h_ref, w_ref,
                   bg_ref, u_ref, inv_ref):
    prev = prev_ref[...]
    M = prev.shape[0]
    gd = jnp.sum(h_ref[...] * w_ref[...], axis=0, keepdims=True)
    gate = jax.nn.sigmoid(gd + bg_ref[0, 0] + 1.0)
    u = gate * (fil_ref[...] - prev) * (1.0 / M)
    pu = jnp.sum(prev * u, axis=0, keepdims=True)
    s1 = jnp.sum(r1_ref[...] * r1_ref[...], axis=0, keepdims=True)
    s2 = jnp.sum(r2_ref[...] * r2_ref[...], axis=0, keepdims=True)
    su = jnp.sum(u * u, axis=0, keepdims=True)
    n2 = sq_ref[...] + 2.0 * pu + s1 * s2 * su
    inv = jnp.minimum(jax.lax.rsqrt(n2), 1.0)
    u_ref[...] = u
    inv_ref[...] = inv


def _update_body(mem_ref, r1_ref, r2_ref, u_ref, inv_ref, out_ref):
    slab = mem_ref[...]
    a = r2_ref[...][:, None, :] * u_ref[...][None, :, :]
    r1b = r1_ref[...][:, None]
    out_ref[...] = (slab + r1b * a[None]) * inv_ref[...]


def kernel(memory_state, hidden_state, role1, role2, filer, W_gate, b_gate):
    B, M = memory_state.shape[0], memory_state.shape[1]
    H = hidden_state.shape[1]
    f32 = jnp.float32

    mem_t = jnp.transpose(memory_state, (1, 2, 3, 0))
    r1t = jnp.transpose(role1)
    r2t = jnp.transpose(role2)
    fil_t = jnp.transpose(filer)
    r1t3 = r1t.reshape(M, 1, B)
    h_t = jnp.transpose(hidden_state)
    wcol = W_gate.reshape(H, 1)
    bg = b_gate.reshape(1, 1)

    R = 4
    prev, sqp = pl.pallas_call(
        _reduce_body,
        grid=(M // R,),
        in_specs=[
            pl.BlockSpec((R, M, M, B), lambda rr: (rr, 0, 0, 0)),
            pl.BlockSpec((R, 1, B), lambda rr: (rr, 0, 0)),
            pl.BlockSpec((M, B), lambda rr: (0, 0)),
        ],
        out_specs=[
            pl.BlockSpec((M, B), lambda rr: (0, 0)),
            pl.BlockSpec((1, B), lambda rr: (0, 0)),
        ],
        out_shape=[
            jax.ShapeDtypeStruct((M, B), f32),
            jax.ShapeDtypeStruct((1, B), f32),
        ],
        compiler_params=pltpu.CompilerParams(
            dimension_semantics=("arbitrary",),
            vmem_limit_bytes=56 * 1024 * 1024,
        ),
    )(mem_t, r1t3, r2t)

    u, inv = pl.pallas_call(
        _finalize_body,
        grid=(1,),
        in_specs=[
            pl.BlockSpec((M, B), lambda i: (0, 0)),
            pl.BlockSpec((1, B), lambda i: (0, 0)),
            pl.BlockSpec((M, B), lambda i: (0, 0)),
            pl.BlockSpec((M, B), lambda i: (0, 0)),
            pl.BlockSpec((M, B), lambda i: (0, 0)),
            pl.BlockSpec((H, B), lambda i: (0, 0)),
            pl.BlockSpec((H, 1), lambda i: (0, 0)),
            pl.BlockSpec((1, 1), lambda i: (0, 0)),
        ],
        out_specs=[
            pl.BlockSpec((M, B), lambda i: (0, 0)),
            pl.BlockSpec((1, B), lambda i: (0, 0)),
        ],
        out_shape=[
            jax.ShapeDtypeStruct((M, B), f32),
            jax.ShapeDtypeStruct((1, B), f32),
        ],
        compiler_params=pltpu.CompilerParams(
            dimension_semantics=("arbitrary",),
        ),
    )(prev, sqp, r1t, r2t, fil_t, h_t, wcol, bg)

    R2 = 2
    out_t = pl.pallas_call(
        _update_body,
        grid=(M // R2,),
        in_specs=[
            pl.BlockSpec((R2, M, M, B), lambda r: (r, 0, 0, 0)),
            pl.BlockSpec((R2, 1, B), lambda r: (r, 0, 0)),
            pl.BlockSpec((M, B), lambda r: (0, 0)),
            pl.BlockSpec((M, B), lambda r: (0, 0)),
            pl.BlockSpec((1, B), lambda r: (0, 0)),
        ],
        out_specs=pl.BlockSpec((R2, M, M, B), lambda r: (r, 0, 0, 0)),
        out_shape=jax.ShapeDtypeStruct((M, M, M, B), f32),
        compiler_params=pltpu.CompilerParams(
            dimension_semantics=("parallel",),
            vmem_limit_bytes=56 * 1024 * 1024,
        ),
    )(mem_t, r1t3, r2t, u, inv)

    return jnp.transpose(out_t, (3, 0, 1, 2))


# speedup vs baseline: 5.2295x; 1.0147x over previous
import jax
import jax.numpy as jnp
from jax.experimental import pallas as pl
from jax.experimental.pallas import tpu as pltpu


def _reduce_body(mem_ref, r1_ref, r2_ref, fil_ref, h_ref, w_ref, bg_ref,
                 u_ref, inv_ref, prev_ref, sq_ref, s1_ref):
    rr = pl.program_id(0)
    nsteps = pl.num_programs(0)
    slab = mem_ref[...]
    r1b = r1_ref[...]
    r2b = r2_ref[...][:, None, :]
    q = jnp.sum(slab * r2b[None], axis=1)
    contrib = jnp.sum(q * r1b, axis=0)
    sqc = jnp.sum(slab * slab, axis=(0, 1, 2), keepdims=True)[0, 0]
    s1c = jnp.sum(r1b * r1b, axis=0)

    @pl.when(rr == 0)
    def _():
        prev_ref[...] = contrib
        sq_ref[...] = sqc
        s1_ref[...] = s1c
    @pl.when(rr != 0)
    def _():
        prev_ref[...] += contrib
        sq_ref[...] += sqc
        s1_ref[...] += s1c

    @pl.when(rr == nsteps - 1)
    def _():
        prev = prev_ref[...]
        M = prev.shape[0]
        gd = jax.lax.dot_general(w_ref[...], h_ref[...],
                                 (((1,), (1,)), ((), ())),
                                 preferred_element_type=jnp.float32)
        gate = jax.nn.sigmoid(gd + bg_ref[0, 0] + 1.0)
        u = gate * (fil_ref[...] - prev) * (1.0 / M)
        pu = jnp.sum(prev * u, axis=0, keepdims=True)
        r2f = r2_ref[...]
        s2 = jnp.sum(r2f * r2f, axis=0, keepdims=True)
        su = jnp.sum(u * u, axis=0, keepdims=True)
        n2 = sq_ref[...] + 2.0 * pu + s1_ref[...] * s2 * su
        u_ref[...] = u
        inv_ref[...] = jnp.minimum(jax.lax.rsqrt(n2), 1.0)


def _update_body(mem_ref, r1_ref, r2_ref, u_ref, inv_ref, out_ref):
    slab = mem_ref[...]
    a = r2_ref[...][:, None, :] * u_ref[...][None, :, :]
    r1b = r1_ref[...][:, None]
    out_ref[...] = (slab + r1b * a[None]) * inv_ref[...]


def kernel(memory_state, hidden_state, role1, role2, filer, W_gate, b_gate):
    B, M = memory_state.shape[0], memory_state.shape[1]
    H = hidden_state.shape[1]
    f32 = jnp.float32

    mem_t = jnp.transpose(memory_state, (1, 2, 3, 0))
    r1t = jnp.transpose(role1)
    r2t = jnp.transpose(role2)
    fil_t = jnp.transpose(filer)
    r1t3 = r1t.reshape(M, 1, B)
    bg = b_gate.reshape(1, 1)

    R = 4
    u, inv = pl.pallas_call(
        _reduce_body,
        grid=(M // R,),
        in_specs=[
            pl.BlockSpec((R, M, M, B), lambda rr: (rr, 0, 0, 0)),
            pl.BlockSpec((R, 1, B), lambda rr: (rr, 0, 0)),
            pl.BlockSpec((M, B), lambda rr: (0, 0)),
            pl.BlockSpec((M, B), lambda rr: (0, 0)),
            pl.BlockSpec((B, H), lambda rr: (0, 0)),
            pl.BlockSpec((1, H), lambda rr: (0, 0)),
            pl.BlockSpec((1, 1), lambda rr: (0, 0)),
        ],
        out_specs=[
            pl.BlockSpec((M, B), lambda rr: (0, 0)),
            pl.BlockSpec((1, B), lambda rr: (0, 0)),
        ],
        out_shape=[
            jax.ShapeDtypeStruct((M, B), f32),
            jax.ShapeDtypeStruct((1, B), f32),
        ],
        scratch_shapes=[
            pltpu.VMEM((M, B), f32),
            pltpu.VMEM((1, B), f32),
            pltpu.VMEM((1, B), f32),
        ],
        compiler_params=pltpu.CompilerParams(
            dimension_semantics=("arbitrary",),
            vmem_limit_bytes=56 * 1024 * 1024,
        ),
    )(mem_t, r1t3, r2t, fil_t, hidden_state, W_gate, bg)

    R2 = 2
    out_t = pl.pallas_call(
        _update_body,
        grid=(M // R2,),
        in_specs=[
            pl.BlockSpec((R2, M, M, B), lambda r: (r, 0, 0, 0)),
            pl.BlockSpec((R2, 1, B), lambda r: (r, 0, 0)),
            pl.BlockSpec((M, B), lambda r: (0, 0)),
            pl.BlockSpec((M, B), lambda r: (0, 0)),
            pl.BlockSpec((1, B), lambda r: (0, 0)),
        ],
        out_specs=pl.BlockSpec((R2, M, M, B), lambda r: (r, 0, 0, 0)),
        out_shape=jax.ShapeDtypeStruct((M, M, M, B), f32),
        compiler_params=pltpu.CompilerParams(
            dimension_semantics=("parallel",),
            vmem_limit_bytes=56 * 1024 * 1024,
        ),
    )(mem_t, r1t3, r2t, u, inv)

    return jnp.transpose(out_t, (3, 0, 1, 2))
